# baseline (speedup 1.0000x reference)
import functools

import jax
import jax.numpy as jnp
from jax import lax
from jax.experimental import pallas as pl
from jax.experimental.pallas import tpu as pltpu
from jax.experimental.pallas import tpu_sc as plsc

N = 10000
E = 320000
F_IN = 128
H = 256
C = 10
B = 64

NC = 2
NS = 16
HH = H // 2
N_PAD = 10240
ROWS_PER_SUB = N_PAD // NS
EDGES_PER_SUB = E // NS
EDGES_PER_TILE = E // (NC * NS)
K = 32
CHUNKS = EDGES_PER_SUB // K
NBUF = 1
TURNS = CHUNKS // NBUF - 1
ZR = 64

ROW_BLK = 1000
NB = N // ROW_BLK

_mesh = plsc.VectorSubcoreMesh(
    core_axis_name="c", subcore_axis_name="s", num_cores=NC, num_subcores=NS
)


@functools.partial(
    pl.kernel,
    out_type=jax.ShapeDtypeStruct((NC, N_PAD, 16), jnp.float32),
    mesh=_mesh,
    scratch_types=[
        pltpu.VMEM((K,), jnp.int32),
        pltpu.VMEM((K, 16), jnp.float32),
        pltpu.VMEM((ZR, 16), jnp.float32),
        pltpu.VMEM_SHARED((N_PAD, 16), jnp.float32),
    ],
)
def _deg_kernel(dst_hbm, out_hbm, dst_v, ones_v, zer_v, acc):
    c = lax.axis_index("c")
    s = lax.axis_index("s")

    @pl.loop(0, K)
    def _(r):
        ones_v[r, pl.ds(0, 16)] = jnp.ones((16,), jnp.float32)

    @pl.loop(0, ZR)
    def _(r):
        zer_v[r, pl.ds(0, 16)] = jnp.zeros((16,), jnp.float32)

    row0 = s * ROWS_PER_SUB

    @pl.loop(0, ROWS_PER_SUB // ZR)
    def _(i):
        pltpu.sync_copy(zer_v, acc.at[pl.ds(row0 + i * ZR, ZR)])

    plsc.subcore_barrier()

    base = (c * NS + s) * EDGES_PER_TILE

    @pl.loop(0, EDGES_PER_TILE // K)
    def _(g):
        pltpu.sync_copy(dst_hbm.at[pl.ds(base + g * K, K)], dst_v)
        pltpu.sync_copy(ones_v, acc.at[dst_v], add=True)

    plsc.subcore_barrier()
    pltpu.sync_copy(
        acc.at[pl.ds(row0, ROWS_PER_SUB)],
        out_hbm.at[c, pl.ds(row0, ROWS_PER_SUB)],
    )


@functools.partial(
    pl.kernel,
    out_type=jax.ShapeDtypeStruct((NC, N_PAD, HH), jnp.float32),
    mesh=_mesh,
    scratch_types=[
        [pltpu.VMEM((K,), jnp.int32) for _ in range(NBUF)],
        [pltpu.VMEM((K,), jnp.int32) for _ in range(NBUF)],
        [pltpu.VMEM((K, HH), jnp.float32) for _ in range(NBUF)],
        pltpu.VMEM_SHARED((N_PAD, HH), jnp.float32),
        [pltpu.SemaphoreType.DMA for _ in range(NBUF)],
        [pltpu.SemaphoreType.DMA for _ in range(NBUF)],
    ],
)
def _scatter_kernel(z_hbm, src_hbm, dst_hbm, out_hbm,
                    sbufs, dbufs, bufs, acc, isems, gsems):
    c = lax.axis_index("c")
    s = lax.axis_index("s")
    ebase = s * EDGES_PER_SUB

    def _idx_start(p, cch):
        pltpu.async_copy(src_hbm.at[pl.ds(ebase + cch * K, K)],
                         sbufs[p], isems[p])
        pltpu.async_copy(dst_hbm.at[pl.ds(ebase + cch * K, K)],
                         dbufs[p], isems[p])

    def _idx_wait(p, cch):
        pltpu.make_async_copy(src_hbm.at[pl.ds(ebase + cch * K, K)],
                              sbufs[p], isems[p]).wait()
        pltpu.make_async_copy(dst_hbm.at[pl.ds(ebase + cch * K, K)],
                              dbufs[p], isems[p]).wait()

    @pl.loop(0, K)
    def _(r):
        @pl.loop(0, HH, step=16)
        def _(j):
            bufs[0][r, pl.ds(j, 16)] = jnp.zeros((16,), jnp.float32)

    row0 = s * ROWS_PER_SUB

    @pl.loop(0, ROWS_PER_SUB // K)
    def _(i):
        pltpu.sync_copy(bufs[0], acc.at[pl.ds(row0 + i * K, K)])

    plsc.subcore_barrier()

    zc = z_hbm.at[c]

    for p in range(NBUF):
        _idx_start(p, p)
    for p in range(NBUF):
        _idx_wait(p, p)
        pltpu.async_copy(zc.at[sbufs[p]], bufs[p], gsems[p])

    @pl.loop(0, TURNS)
    def _(q):
        base = q * NBUF
        for p in range(NBUF):
            cch = base + p
            pltpu.make_async_copy(zc.at[sbufs[p]], bufs[p], gsems[p]).wait()
            pltpu.sync_copy(bufs[p], acc.at[dbufs[p]], add=True)
            _idx_start(p, cch + NBUF)
            _idx_wait(p, cch + NBUF)
            pltpu.async_copy(zc.at[sbufs[p]], bufs[p], gsems[p])

    for p in range(NBUF):
        pltpu.make_async_copy(zc.at[sbufs[p]], bufs[p], gsems[p]).wait()
        pltpu.sync_copy(bufs[p], acc.at[dbufs[p]], add=True)

    plsc.subcore_barrier()
    pltpu.sync_copy(
        acc.at[pl.ds(row0, ROWS_PER_SUB)],
        out_hbm.at[c, pl.ds(row0, ROWS_PER_SUB)],
    )


def _dot(a, b):
    return lax.dot_general(a, b, (((1,), (0,)), ((), ())),
                           precision=lax.Precision.HIGHEST)


def _dinv_of(d_ref):
    deg = d_ref[0, :, 0:1] + d_ref[1, :, 0:1] + 1.0
    return lax.rsqrt(deg)


def _mm_body(x_ref, w_ref, o_ref):
    o_ref[...] = _dot(x_ref[...], w_ref[...])


def _split_body(y_ref, d_ref, o_ref):
    z = y_ref[...] * _dinv_of(d_ref)
    o_ref[0] = z[:, :HH]
    o_ref[1] = z[:, HH:]


def _elu(p):
    return jnp.where(p > 0, p, jnp.exp(jnp.minimum(p, 0.0)) - 1.0)


def _layer_body(a_ref, z_ref, d_ref, b_ref, w_ref, o_ref):
    dinv = _dinv_of(d_ref)
    agg = jnp.concatenate([a_ref[0] + z_ref[0], a_ref[1] + z_ref[1]], axis=1)
    h = _elu(dinv * agg + b_ref[...])
    zn = _dot(h, w_ref[...]) * dinv
    o_ref[0] = zn[:, :HH]
    o_ref[1] = zn[:, HH:]


def _final_body(a_ref, z_ref, d_ref, b_ref, bat_ref, wl_ref, bl_ref,
                o_ref, g_acc):
    i = pl.program_id(0)

    @pl.when(i == 0)
    def _():
        g_acc[...] = jnp.full((B, H), -jnp.inf, jnp.float32)

    dinv = _dinv_of(d_ref)
    agg = jnp.concatenate([a_ref[0] + z_ref[0], a_ref[1] + z_ref[1]], axis=1)
    h = _elu(dinv * agg + b_ref[...])

    bat = bat_ref[...]
    b_lo = bat_ref[0, 0]
    b_hi = bat_ref[ROW_BLK - 1, 0]

    def seg_body(b, _):
        m = jnp.where(bat == b, h, -jnp.inf)
        cur = g_acc[pl.ds(b, 1), :]
        g_acc[pl.ds(b, 1), :] = jnp.maximum(cur, jnp.max(m, axis=0)[None, :])
        return 0

    lax.fori_loop(b_lo, b_hi + 1, seg_body, 0)

    @pl.when(i == NB - 1)
    def _():
        g = g_acc[...]
        logits = _dot(g, wl_ref[...]) + bl_ref[...]
        col = lax.broadcasted_iota(jnp.int32, (B, 128), 1)
        valid = col < C
        lm = jnp.where(valid, logits, -jnp.inf)
        mx = jnp.max(lm, axis=1, keepdims=True)
        e = jnp.where(valid, jnp.exp(lm - mx), 0.0)
        lse = jnp.log(jnp.sum(e, axis=1, keepdims=True)) + mx
        o_ref[...] = lm - lse


def _mm_call(x, w):
    return pl.pallas_call(
        _mm_body,
        grid=(NB,),
        in_specs=[
            pl.BlockSpec((ROW_BLK, F_IN), lambda i: (i, 0)),
            pl.BlockSpec((F_IN, H), lambda i: (0, 0)),
        ],
        out_specs=pl.BlockSpec((ROW_BLK, H), lambda i: (i, 0)),
        out_shape=jax.ShapeDtypeStruct((N, H), jnp.float32),
    )(x, w)


def _split_call(y, deg16):
    return pl.pallas_call(
        _split_body,
        grid=(NB,),
        in_specs=[
            pl.BlockSpec((ROW_BLK, H), lambda i: (i, 0)),
            pl.BlockSpec((NC, ROW_BLK, 16), lambda i: (0, i, 0)),
        ],
        out_specs=pl.BlockSpec((NC, ROW_BLK, HH), lambda i: (0, i, 0)),
        out_shape=jax.ShapeDtypeStruct((NC, N, HH), jnp.float32),
    )(y, deg16)


def _layer_call(a, z, deg16, b2d, w):
    return pl.pallas_call(
        _layer_body,
        grid=(NB,),
        in_specs=[
            pl.BlockSpec((NC, ROW_BLK, HH), lambda i: (0, i, 0)),
            pl.BlockSpec((NC, ROW_BLK, HH), lambda i: (0, i, 0)),
            pl.BlockSpec((NC, ROW_BLK, 16), lambda i: (0, i, 0)),
            pl.BlockSpec((1, H), lambda i: (0, 0)),
            pl.BlockSpec((H, H), lambda i: (0, 0)),
        ],
        out_specs=pl.BlockSpec((NC, ROW_BLK, HH), lambda i: (0, i, 0)),
        out_shape=jax.ShapeDtypeStruct((NC, N, HH), jnp.float32),
    )(a, z, deg16, b2d, w)


def _final_call(a, z, deg16, b2d, bat3d, wl_pad, bl_pad):
    return pl.pallas_call(
        _final_body,
        grid=(NB,),
        in_specs=[
            pl.BlockSpec((NC, ROW_BLK, HH), lambda i: (0, i, 0)),
            pl.BlockSpec((NC, ROW_BLK, HH), lambda i: (0, i, 0)),
            pl.BlockSpec((NC, ROW_BLK, 16), lambda i: (0, i, 0)),
            pl.BlockSpec((1, H), lambda i: (0, 0)),
            pl.BlockSpec((ROW_BLK, 1), lambda i: (i, 0)),
            pl.BlockSpec((H, 128), lambda i: (0, 0)),
            pl.BlockSpec((1, 128), lambda i: (0, 0)),
        ],
        out_specs=pl.BlockSpec((B, 128), lambda i: (0, 0)),
        out_shape=jax.ShapeDtypeStruct((B, 128), jnp.float32),
        scratch_shapes=[pltpu.VMEM((B, H), jnp.float32)],
    )(a, z, deg16, b2d, bat3d, wl_pad, bl_pad)


def kernel(x, edge_index, batch, W1, b1, W2, b2, W3, b3, Wl, bl):
    src = edge_index[0]
    dst = edge_index[1]

    deg16 = _deg_kernel(dst)
    y1 = _mm_call(x, W1)
    z1 = _split_call(y1, deg16)

    wl_pad = jnp.zeros((H, 128), jnp.float32).at[:, :C].set(Wl)
    bl_pad = jnp.zeros((1, 128), jnp.float32).at[0, :C].set(bl)
    bat2d = batch.reshape(N, 1)

    a1 = _scatter_kernel(z1, src, dst)
    z2 = _layer_call(a1, z1, deg16, b1.reshape(1, H), W2)
    a2 = _scatter_kernel(z2, src, dst)
    z3 = _layer_call(a2, z2, deg16, b2.reshape(1, H), W3)
    a3 = _scatter_kernel(z3, src, dst)
    out = _final_call(a3, z3, deg16, b3.reshape(1, H), bat2d, wl_pad, bl_pad)
    return out[:, :C]


# speedup vs baseline: 2.4915x; 2.4915x over previous
import functools

import jax
import jax.numpy as jnp
from jax import lax
from jax.experimental import pallas as pl
from jax.experimental.pallas import tpu as pltpu
from jax.experimental.pallas import tpu_sc as plsc

N = 10000
E = 320000
F_IN = 128
H = 256
C = 10
B = 64

NC = 2
NS = 16
HH = H // 2
N_PAD = 10240
ROWS_PER_SUB = N_PAD // NS
EDGES_PER_SUB = E // NS
EDGES_PER_TILE = E // (NC * NS)
K = 40
CHUNKS = EDGES_PER_SUB // K
G = 5
GROUPS = CHUNKS // G
PAIRS = GROUPS // 2
ZR = 64

ROW_BLK = 1000
NB = N // ROW_BLK

_mesh = plsc.VectorSubcoreMesh(
    core_axis_name="c", subcore_axis_name="s", num_cores=NC, num_subcores=NS
)


@functools.partial(
    pl.kernel,
    out_type=jax.ShapeDtypeStruct((NC, N_PAD, 16), jnp.float32),
    mesh=_mesh,
    scratch_types=[
        pltpu.VMEM((K,), jnp.int32),
        pltpu.VMEM((K, 16), jnp.float32),
        pltpu.VMEM((ZR, 16), jnp.float32),
        pltpu.VMEM_SHARED((N_PAD, 16), jnp.float32),
    ],
)
def _deg_kernel(dst_hbm, out_hbm, dst_v, ones_v, zer_v, acc):
    c = lax.axis_index("c")
    s = lax.axis_index("s")

    @pl.loop(0, K)
    def _(r):
        ones_v[r, pl.ds(0, 16)] = jnp.ones((16,), jnp.float32)

    @pl.loop(0, ZR)
    def _(r):
        zer_v[r, pl.ds(0, 16)] = jnp.zeros((16,), jnp.float32)

    row0 = s * ROWS_PER_SUB

    @pl.loop(0, ROWS_PER_SUB // ZR)
    def _(i):
        pltpu.sync_copy(zer_v, acc.at[pl.ds(row0 + i * ZR, ZR)])

    plsc.subcore_barrier()

    base = (c * NS + s) * EDGES_PER_TILE

    @pl.loop(0, EDGES_PER_TILE // K)
    def _(g):
        pltpu.sync_copy(dst_hbm.at[pl.ds(base + g * K, K)], dst_v)
        pltpu.sync_copy(ones_v, acc.at[dst_v], add=True)

    plsc.subcore_barrier()
    pltpu.sync_copy(
        acc.at[pl.ds(row0, ROWS_PER_SUB)],
        out_hbm.at[c, pl.ds(row0, ROWS_PER_SUB)],
    )


@functools.partial(
    pl.kernel,
    out_type=jax.ShapeDtypeStruct((NC, N_PAD, HH), jnp.float32),
    mesh=_mesh,
    scratch_types=[
        [[pltpu.VMEM((K,), jnp.int32) for _ in range(G)] for _ in range(2)],
        [[pltpu.VMEM((K,), jnp.int32) for _ in range(G)] for _ in range(2)],
        [pltpu.VMEM((K, HH), jnp.float32) for _ in range(G)],
        pltpu.VMEM_SHARED((N_PAD, HH), jnp.float32),
        [pltpu.SemaphoreType.DMA for _ in range(2)],
        pltpu.SemaphoreType.DMA,
    ],
)
def _scatter_kernel(z_hbm, src_hbm, dst_hbm, out_hbm,
                    sbufs, dbufs, bufs, acc, isems, gsem):
    c = lax.axis_index("c")
    s = lax.axis_index("s")
    ebase = s * EDGES_PER_SUB

    def _idx_fire(t, grp):
        for p in range(G):
            off = ebase + (grp * G + p) * K
            pltpu.async_copy(src_hbm.at[pl.ds(off, K)], sbufs[t][p], isems[t])
            pltpu.async_copy(dst_hbm.at[pl.ds(off, K)], dbufs[t][p], isems[t])

    def _idx_drain(t, grp):
        for p in range(G):
            off = ebase + (grp * G + p) * K
            pltpu.make_async_copy(
                src_hbm.at[pl.ds(off, K)], sbufs[t][p], isems[t]).wait()
            pltpu.make_async_copy(
                dst_hbm.at[pl.ds(off, K)], dbufs[t][p], isems[t]).wait()

    @pl.loop(0, K)
    def _(r):
        @pl.loop(0, HH, step=16)
        def _(j):
            bufs[0][r, pl.ds(j, 16)] = jnp.zeros((16,), jnp.float32)

    row0 = s * ROWS_PER_SUB

    @pl.loop(0, ROWS_PER_SUB // K)
    def _(i):
        pltpu.sync_copy(bufs[0], acc.at[pl.ds(row0 + i * K, K)])

    plsc.subcore_barrier()

    zc = z_hbm.at[c]

    _idx_fire(0, 0)

    @pl.loop(0, PAIRS)
    def _(q):
        for t in range(2):
            grp = 2 * q + t
            _idx_drain(t, grp)
            for p in range(G):
                pltpu.async_copy(zc.at[sbufs[t][p]], bufs[p], gsem)
            if t == 0:
                _idx_fire(1, grp + 1)
            else:
                @pl.when(q < PAIRS - 1)
                def _():
                    _idx_fire(0, grp + 1)
            for p in range(G):
                pltpu.make_async_copy(zc.at[sbufs[t][p]], bufs[p], gsem).wait()
            for p in range(G):
                pltpu.sync_copy(bufs[p], acc.at[dbufs[t][p]], add=True)

    plsc.subcore_barrier()
    pltpu.sync_copy(
        acc.at[pl.ds(row0, ROWS_PER_SUB)],
        out_hbm.at[c, pl.ds(row0, ROWS_PER_SUB)],
    )


def _dot(a, b):
    return lax.dot_general(a, b, (((1,), (0,)), ((), ())),
                           precision=lax.Precision.HIGHEST)


def _dinv_of(d_ref):
    deg = d_ref[0, :, 0:1] + d_ref[1, :, 0:1] + 1.0
    return lax.rsqrt(deg)


def _mm_body(x_ref, w_ref, o_ref):
    o_ref[...] = _dot(x_ref[...], w_ref[...])


def _split_body(y_ref, d_ref, o_ref):
    z = y_ref[...] * _dinv_of(d_ref)
    o_ref[0] = z[:, :HH]
    o_ref[1] = z[:, HH:]


def _elu(p):
    return jnp.where(p > 0, p, jnp.exp(jnp.minimum(p, 0.0)) - 1.0)


def _layer_body(a_ref, z_ref, d_ref, b_ref, w_ref, o_ref):
    dinv = _dinv_of(d_ref)
    agg = jnp.concatenate([a_ref[0] + z_ref[0], a_ref[1] + z_ref[1]], axis=1)
    h = _elu(dinv * agg + b_ref[...])
    zn = _dot(h, w_ref[...]) * dinv
    o_ref[0] = zn[:, :HH]
    o_ref[1] = zn[:, HH:]


def _final_body(a_ref, z_ref, d_ref, b_ref, bat_ref, wl_ref, bl_ref,
                o_ref, g_acc):
    i = pl.program_id(0)

    @pl.when(i == 0)
    def _():
        g_acc[...] = jnp.full((B, H), -jnp.inf, jnp.float32)

    dinv = _dinv_of(d_ref)
    agg = jnp.concatenate([a_ref[0] + z_ref[0], a_ref[1] + z_ref[1]], axis=1)
    h = _elu(dinv * agg + b_ref[...])

    bat = bat_ref[...]
    b_lo = bat_ref[0, 0]
    b_hi = bat_ref[ROW_BLK - 1, 0]

    def seg_body(b, _):
        m = jnp.where(bat == b, h, -jnp.inf)
        cur = g_acc[pl.ds(b, 1), :]
        g_acc[pl.ds(b, 1), :] = jnp.maximum(cur, jnp.max(m, axis=0)[None, :])
        return 0

    lax.fori_loop(b_lo, b_hi + 1, seg_body, 0)

    @pl.when(i == NB - 1)
    def _():
        g = g_acc[...]
        logits = _dot(g, wl_ref[...]) + bl_ref[...]
        col = lax.broadcasted_iota(jnp.int32, (B, 128), 1)
        valid = col < C
        lm = jnp.where(valid, logits, -jnp.inf)
        mx = jnp.max(lm, axis=1, keepdims=True)
        e = jnp.where(valid, jnp.exp(lm - mx), 0.0)
        lse = jnp.log(jnp.sum(e, axis=1, keepdims=True)) + mx
        o_ref[...] = lm - lse


def _mm_call(x, w):
    return pl.pallas_call(
        _mm_body,
        grid=(NB,),
        in_specs=[
            pl.BlockSpec((ROW_BLK, F_IN), lambda i: (i, 0)),
            pl.BlockSpec((F_IN, H), lambda i: (0, 0)),
        ],
        out_specs=pl.BlockSpec((ROW_BLK, H), lambda i: (i, 0)),
        out_shape=jax.ShapeDtypeStruct((N, H), jnp.float32),
    )(x, w)


def _split_call(y, deg16):
    return pl.pallas_call(
        _split_body,
        grid=(NB,),
        in_specs=[
            pl.BlockSpec((ROW_BLK, H), lambda i: (i, 0)),
            pl.BlockSpec((NC, ROW_BLK, 16), lambda i: (0, i, 0)),
        ],
        out_specs=pl.BlockSpec((NC, ROW_BLK, HH), lambda i: (0, i, 0)),
        out_shape=jax.ShapeDtypeStruct((NC, N, HH), jnp.float32),
    )(y, deg16)


def _layer_call(a, z, deg16, b2d, w):
    return pl.pallas_call(
        _layer_body,
        grid=(NB,),
        in_specs=[
            pl.BlockSpec((NC, ROW_BLK, HH), lambda i: (0, i, 0)),
            pl.BlockSpec((NC, ROW_BLK, HH), lambda i: (0, i, 0)),
            pl.BlockSpec((NC, ROW_BLK, 16), lambda i: (0, i, 0)),
            pl.BlockSpec((1, H), lambda i: (0, 0)),
            pl.BlockSpec((H, H), lambda i: (0, 0)),
        ],
        out_specs=pl.BlockSpec((NC, ROW_BLK, HH), lambda i: (0, i, 0)),
        out_shape=jax.ShapeDtypeStruct((NC, N, HH), jnp.float32),
    )(a, z, deg16, b2d, w)


def _final_call(a, z, deg16, b2d, bat3d, wl_pad, bl_pad):
    return pl.pallas_call(
        _final_body,
        grid=(NB,),
        in_specs=[
            pl.BlockSpec((NC, ROW_BLK, HH), lambda i: (0, i, 0)),
            pl.BlockSpec((NC, ROW_BLK, HH), lambda i: (0, i, 0)),
            pl.BlockSpec((NC, ROW_BLK, 16), lambda i: (0, i, 0)),
            pl.BlockSpec((1, H), lambda i: (0, 0)),
            pl.BlockSpec((ROW_BLK, 1), lambda i: (i, 0)),
            pl.BlockSpec((H, 128), lambda i: (0, 0)),
            pl.BlockSpec((1, 128), lambda i: (0, 0)),
        ],
        out_specs=pl.BlockSpec((B, 128), lambda i: (0, 0)),
        out_shape=jax.ShapeDtypeStruct((B, 128), jnp.float32),
        scratch_shapes=[pltpu.VMEM((B, H), jnp.float32)],
    )(a, z, deg16, b2d, bat3d, wl_pad, bl_pad)


def kernel(x, edge_index, batch, W1, b1, W2, b2, W3, b3, Wl, bl):
    src = edge_index[0]
    dst = edge_index[1]

    deg16 = _deg_kernel(dst)
    y1 = _mm_call(x, W1)
    z1 = _split_call(y1, deg16)

    wl_pad = jnp.zeros((H, 128), jnp.float32).at[:, :C].set(Wl)
    bl_pad = jnp.zeros((1, 128), jnp.float32).at[0, :C].set(bl)
    bat2d = batch.reshape(N, 1)

    a1 = _scatter_kernel(z1, src, dst)
    z2 = _layer_call(a1, z1, deg16, b1.reshape(1, H), W2)
    a2 = _scatter_kernel(z2, src, dst)
    z3 = _layer_call(a2, z2, deg16, b2.reshape(1, H), W3)
    a3 = _scatter_kernel(z3, src, dst)
    out = _final_call(a3, z3, deg16, b3.reshape(1, H), bat2d, wl_pad, bl_pad)
    return out[:, :C]


# speedup vs baseline: 2.7153x; 1.0898x over previous
import functools

import jax
import jax.numpy as jnp
from jax import lax
from jax.experimental import pallas as pl
from jax.experimental.pallas import tpu as pltpu
from jax.experimental.pallas import tpu_sc as plsc

N = 10000
E = 320000
F_IN = 128
H = 256
C = 10
B = 64

NC = 2
NS = 16
HH = H // 2
N_PAD = 10240
ROWS_PER_SUB = N_PAD // NS
EDGES_PER_SUB = E // NS
EDGES_PER_TILE = E // (NC * NS)
K = 40
CHUNKS = EDGES_PER_SUB // K
G = 5
GROUPS = CHUNKS // G
PAIRS = GROUPS // 2
ZR = 64

ROW_BLK = 1000
NB = N // ROW_BLK

_mesh = plsc.VectorSubcoreMesh(
    core_axis_name="c", subcore_axis_name="s", num_cores=NC, num_subcores=NS
)


@functools.partial(
    pl.kernel,
    out_type=jax.ShapeDtypeStruct((NC, N_PAD, 16), jnp.float32),
    mesh=_mesh,
    scratch_types=[
        [[pltpu.VMEM((K,), jnp.int32) for _ in range(G)] for _ in range(2)],
        pltpu.VMEM((K, 16), jnp.float32),
        pltpu.VMEM((ZR, 16), jnp.float32),
        pltpu.VMEM_SHARED((N_PAD, 16), jnp.float32),
        [pltpu.SemaphoreType.DMA for _ in range(2)],
    ],
)
def _deg_kernel(dst_hbm, out_hbm, dbufs, ones_v, zer_v, acc, isems):
    c = lax.axis_index("c")
    s = lax.axis_index("s")
    ebase = (c * NS + s) * EDGES_PER_TILE
    dgroups = EDGES_PER_TILE // (G * K)
    dpairs = dgroups // 2

    def _idx_fire(t, grp):
        for p in range(G):
            off = ebase + (grp * G + p) * K
            pltpu.async_copy(dst_hbm.at[pl.ds(off, K)], dbufs[t][p], isems[t])

    def _idx_drain(t, grp):
        for p in range(G):
            off = ebase + (grp * G + p) * K
            pltpu.make_async_copy(
                dst_hbm.at[pl.ds(off, K)], dbufs[t][p], isems[t]).wait()

    @pl.loop(0, K)
    def _(r):
        ones_v[r, pl.ds(0, 16)] = jnp.ones((16,), jnp.float32)

    @pl.loop(0, ZR)
    def _(r):
        zer_v[r, pl.ds(0, 16)] = jnp.zeros((16,), jnp.float32)

    row0 = s * ROWS_PER_SUB

    @pl.loop(0, ROWS_PER_SUB // ZR)
    def _(i):
        pltpu.sync_copy(zer_v, acc.at[pl.ds(row0 + i * ZR, ZR)])

    plsc.subcore_barrier()

    _idx_fire(0, 0)

    @pl.loop(0, dpairs)
    def _(q):
        for t in range(2):
            grp = 2 * q + t
            _idx_drain(t, grp)
            if t == 0:
                _idx_fire(1, grp + 1)
            else:
                @pl.when(q < dpairs - 1)
                def _():
                    _idx_fire(0, grp + 1)
            for p in range(G):
                pltpu.sync_copy(ones_v, acc.at[dbufs[t][p]], add=True)

    plsc.subcore_barrier()
    pltpu.sync_copy(
        acc.at[pl.ds(row0, ROWS_PER_SUB)],
        out_hbm.at[c, pl.ds(row0, ROWS_PER_SUB)],
    )


@functools.partial(
    pl.kernel,
    out_type=jax.ShapeDtypeStruct((NC, N_PAD, HH), jnp.float32),
    mesh=_mesh,
    scratch_types=[
        [[pltpu.VMEM((K,), jnp.int32) for _ in range(G)] for _ in range(2)],
        [[pltpu.VMEM((K,), jnp.int32) for _ in range(G)] for _ in range(2)],
        [pltpu.VMEM((K, HH), jnp.float32) for _ in range(G)],
        pltpu.VMEM_SHARED((N_PAD, HH), jnp.float32),
        [pltpu.SemaphoreType.DMA for _ in range(2)],
        pltpu.SemaphoreType.DMA,
    ],
)
def _scatter_kernel(z_hbm, src_hbm, dst_hbm, out_hbm,
                    sbufs, dbufs, bufs, acc, isems, gsem):
    c = lax.axis_index("c")
    s = lax.axis_index("s")
    ebase = s * EDGES_PER_SUB

    def _idx_fire(t, grp):
        for p in range(G):
            off = ebase + (grp * G + p) * K
            pltpu.async_copy(src_hbm.at[pl.ds(off, K)], sbufs[t][p], isems[t])
            pltpu.async_copy(dst_hbm.at[pl.ds(off, K)], dbufs[t][p], isems[t])

    def _idx_drain(t, grp):
        for p in range(G):
            off = ebase + (grp * G + p) * K
            pltpu.make_async_copy(
                src_hbm.at[pl.ds(off, K)], sbufs[t][p], isems[t]).wait()
            pltpu.make_async_copy(
                dst_hbm.at[pl.ds(off, K)], dbufs[t][p], isems[t]).wait()

    @pl.loop(0, K)
    def _(r):
        @pl.loop(0, HH, step=16)
        def _(j):
            bufs[0][r, pl.ds(j, 16)] = jnp.zeros((16,), jnp.float32)

    row0 = s * ROWS_PER_SUB

    @pl.loop(0, ROWS_PER_SUB // K)
    def _(i):
        pltpu.sync_copy(bufs[0], acc.at[pl.ds(row0 + i * K, K)])

    plsc.subcore_barrier()

    zc = z_hbm.at[c]

    _idx_fire(0, 0)

    @pl.loop(0, PAIRS)
    def _(q):
        for t in range(2):
            grp = 2 * q + t
            _idx_drain(t, grp)
            for p in range(G):
                pltpu.async_copy(zc.at[sbufs[t][p]], bufs[p], gsem)
            if t == 0:
                _idx_fire(1, grp + 1)
            else:
                @pl.when(q < PAIRS - 1)
                def _():
                    _idx_fire(0, grp + 1)
            for p in range(G):
                pltpu.make_async_copy(zc.at[sbufs[t][p]], bufs[p], gsem).wait()
            for p in range(G):
                pltpu.sync_copy(bufs[p], acc.at[dbufs[t][p]], add=True)

    plsc.subcore_barrier()
    pltpu.sync_copy(
        acc.at[pl.ds(row0, ROWS_PER_SUB)],
        out_hbm.at[c, pl.ds(row0, ROWS_PER_SUB)],
    )


def _dot(a, b):
    return lax.dot_general(a, b, (((1,), (0,)), ((), ())),
                           precision=lax.Precision.HIGHEST)


def _dinv_of(d_ref):
    deg = d_ref[0, :, 0:1] + d_ref[1, :, 0:1] + 1.0
    return lax.rsqrt(deg)


def _mm_body(x_ref, w_ref, o_ref):
    o_ref[...] = _dot(x_ref[...], w_ref[...])


def _split_body(y_ref, d_ref, o_ref):
    z = y_ref[...] * _dinv_of(d_ref)
    o_ref[0] = z[:, :HH]
    o_ref[1] = z[:, HH:]


def _elu(p):
    return jnp.where(p > 0, p, jnp.exp(jnp.minimum(p, 0.0)) - 1.0)


def _layer_body(a_ref, z_ref, d_ref, b_ref, w_ref, o_ref):
    dinv = _dinv_of(d_ref)
    agg = jnp.concatenate([a_ref[0] + z_ref[0], a_ref[1] + z_ref[1]], axis=1)
    h = _elu(dinv * agg + b_ref[...])
    zn = _dot(h, w_ref[...]) * dinv
    o_ref[0] = zn[:, :HH]
    o_ref[1] = zn[:, HH:]


def _final_body(a_ref, z_ref, d_ref, b_ref, bat_ref, wl_ref, bl_ref,
                o_ref, g_acc):
    i = pl.program_id(0)

    @pl.when(i == 0)
    def _():
        g_acc[...] = jnp.full((B, H), -jnp.inf, jnp.float32)

    dinv = _dinv_of(d_ref)
    agg = jnp.concatenate([a_ref[0] + z_ref[0], a_ref[1] + z_ref[1]], axis=1)
    h = _elu(dinv * agg + b_ref[...])

    bat = bat_ref[...]
    b_lo = bat_ref[0, 0]
    b_hi = bat_ref[ROW_BLK - 1, 0]

    def seg_body(b, _):
        m = jnp.where(bat == b, h, -jnp.inf)
        cur = g_acc[pl.ds(b, 1), :]
        g_acc[pl.ds(b, 1), :] = jnp.maximum(cur, jnp.max(m, axis=0)[None, :])
        return 0

    lax.fori_loop(b_lo, b_hi + 1, seg_body, 0)

    @pl.when(i == NB - 1)
    def _():
        g = g_acc[...]
        logits = _dot(g, wl_ref[...]) + bl_ref[...]
        col = lax.broadcasted_iota(jnp.int32, (B, 128), 1)
        valid = col < C
        lm = jnp.where(valid, logits, -jnp.inf)
        mx = jnp.max(lm, axis=1, keepdims=True)
        e = jnp.where(valid, jnp.exp(lm - mx), 0.0)
        lse = jnp.log(jnp.sum(e, axis=1, keepdims=True)) + mx
        o_ref[...] = lm - lse


def _mm_call(x, w):
    return pl.pallas_call(
        _mm_body,
        grid=(NB,),
        in_specs=[
            pl.BlockSpec((ROW_BLK, F_IN), lambda i: (i, 0)),
            pl.BlockSpec((F_IN, H), lambda i: (0, 0)),
        ],
        out_specs=pl.BlockSpec((ROW_BLK, H), lambda i: (i, 0)),
        out_shape=jax.ShapeDtypeStruct((N, H), jnp.float32),
    )(x, w)


def _split_call(y, deg16):
    return pl.pallas_call(
        _split_body,
        grid=(NB,),
        in_specs=[
            pl.BlockSpec((ROW_BLK, H), lambda i: (i, 0)),
            pl.BlockSpec((NC, ROW_BLK, 16), lambda i: (0, i, 0)),
        ],
        out_specs=pl.BlockSpec((NC, ROW_BLK, HH), lambda i: (0, i, 0)),
        out_shape=jax.ShapeDtypeStruct((NC, N, HH), jnp.float32),
    )(y, deg16)


def _layer_call(a, z, deg16, b2d, w):
    return pl.pallas_call(
        _layer_body,
        grid=(NB,),
        in_specs=[
            pl.BlockSpec((NC, ROW_BLK, HH), lambda i: (0, i, 0)),
            pl.BlockSpec((NC, ROW_BLK, HH), lambda i: (0, i, 0)),
            pl.BlockSpec((NC, ROW_BLK, 16), lambda i: (0, i, 0)),
            pl.BlockSpec((1, H), lambda i: (0, 0)),
            pl.BlockSpec((H, H), lambda i: (0, 0)),
        ],
        out_specs=pl.BlockSpec((NC, ROW_BLK, HH), lambda i: (0, i, 0)),
        out_shape=jax.ShapeDtypeStruct((NC, N, HH), jnp.float32),
    )(a, z, deg16, b2d, w)


def _final_call(a, z, deg16, b2d, bat3d, wl_pad, bl_pad):
    return pl.pallas_call(
        _final_body,
        grid=(NB,),
        in_specs=[
            pl.BlockSpec((NC, ROW_BLK, HH), lambda i: (0, i, 0)),
            pl.BlockSpec((NC, ROW_BLK, HH), lambda i: (0, i, 0)),
            pl.BlockSpec((NC, ROW_BLK, 16), lambda i: (0, i, 0)),
            pl.BlockSpec((1, H), lambda i: (0, 0)),
            pl.BlockSpec((ROW_BLK, 1), lambda i: (i, 0)),
            pl.BlockSpec((H, 128), lambda i: (0, 0)),
            pl.BlockSpec((1, 128), lambda i: (0, 0)),
        ],
        out_specs=pl.BlockSpec((B, 128), lambda i: (0, 0)),
        out_shape=jax.ShapeDtypeStruct((B, 128), jnp.float32),
        scratch_shapes=[pltpu.VMEM((B, H), jnp.float32)],
    )(a, z, deg16, b2d, bat3d, wl_pad, bl_pad)


def kernel(x, edge_index, batch, W1, b1, W2, b2, W3, b3, Wl, bl):
    src = edge_index[0]
    dst = edge_index[1]

    deg16 = _deg_kernel(dst)
    y1 = _mm_call(x, W1)
    z1 = _split_call(y1, deg16)

    wl_pad = jnp.zeros((H, 128), jnp.float32).at[:, :C].set(Wl)
    bl_pad = jnp.zeros((1, 128), jnp.float32).at[0, :C].set(bl)
    bat2d = batch.reshape(N, 1)

    a1 = _scatter_kernel(z1, src, dst)
    z2 = _layer_call(a1, z1, deg16, b1.reshape(1, H), W2)
    a2 = _scatter_kernel(z2, src, dst)
    z3 = _layer_call(a2, z2, deg16, b2.reshape(1, H), W3)
    a3 = _scatter_kernel(z3, src, dst)
    out = _final_call(a3, z3, deg16, b3.reshape(1, H), bat2d, wl_pad, bl_pad)
    return out[:, :C]
